# baseline (speedup 1.0000x reference)
import functools

import jax
import jax.numpy as jnp
from jax import lax
from jax.experimental import pallas as pl
from jax.experimental.pallas import tpu as pltpu
from jax.experimental.pallas import tpu_sc as plsc

N = 10000
NPAD = 10240
NW = 32
EPER = 10240
EPAD = NW * EPER
TROWS = NPAD // 16
DEGW = 16
KDEG = 128

_MESH = plsc.VectorSubcoreMesh(
    core_axis_name="c", subcore_axis_name="s", num_cores=2, num_subcores=16
)

_SC_PARAMS = pltpu.CompilerParams(use_tc_tiling_on_sc=False)


@functools.partial(
    pl.kernel,
    out_type=jax.ShapeDtypeStruct((2, NPAD, DEGW), jnp.float32),
    mesh=_MESH,
    scratch_types=[
        pltpu.VMEM((EPER // KDEG, KDEG), jnp.int32),
        pltpu.VMEM((KDEG, DEGW), jnp.float32),
        pltpu.VMEM_SHARED((NPAD, DEGW), jnp.float32),
    ],
    compiler_params=_SC_PARAMS,
)
def _deg_kernel(ec_hbm, ones_hbm, zz_hbm, out_hbm, col_v, obuf, acc):
    cc = lax.axis_index("c")
    ss = lax.axis_index("s")
    wid = cc * 16 + ss
    pltpu.sync_copy(zz_hbm.at[pl.ds(ss * TROWS, TROWS)],
                    acc.at[pl.ds(ss * TROWS, TROWS)])
    pltpu.sync_copy(ones_hbm, obuf)
    pltpu.sync_copy(ec_hbm.at[wid], col_v)
    plsc.subcore_barrier()

    def step(j, carry):
        pltpu.sync_copy(obuf, acc.at[col_v.at[j]], add=True)
        return carry

    lax.fori_loop(0, EPER // KDEG, step, 0)
    plsc.subcore_barrier()
    pltpu.sync_copy(acc.at[pl.ds(ss * TROWS, TROWS)],
                    out_hbm.at[cc, pl.ds(ss * TROWS, TROWS)])


def _make_agg(F):
    K = 128
    CH = EPER // K
    CHH = CH // 2
    NPAIR = CHH // 2

    @functools.partial(
        pl.kernel,
        out_type=jax.ShapeDtypeStruct((2, NPAD, F), jnp.float32),
        mesh=_MESH,
        scratch_types=[
            pltpu.VMEM((CHH, K), jnp.int32),
            pltpu.VMEM((CHH, K), jnp.int32),
            pltpu.VMEM((K, F), jnp.float32),
            pltpu.VMEM((K, F), jnp.float32),
            pltpu.VMEM_SHARED((NPAD, F), jnp.float32),
            pltpu.SemaphoreType.DMA,
            pltpu.SemaphoreType.DMA,
            pltpu.SemaphoreType.DMA,
            pltpu.SemaphoreType.DMA,
        ],
        compiler_params=_SC_PARAMS,
    )
    def agg(h_hbm, er_hbm, ec_hbm, zk_hbm, out_hbm, row_v, col_v, gbufa,
            gbufb, acc, sga, sgb, ssa, ssb):
        cc = lax.axis_index("c")
        ss = lax.axis_index("s")
        wid = cc * 16 + ss
        pltpu.sync_copy(zk_hbm, gbufa)
        for m in range(TROWS // K):
            pltpu.sync_copy(gbufa, acc.at[pl.ds(ss * TROWS + m * K, K)])
        plsc.subcore_barrier()

        for half in range(2):
            pltpu.sync_copy(er_hbm.at[wid, pl.ds(half * CHH, CHH)], row_v)
            pltpu.sync_copy(ec_hbm.at[wid, pl.ds(half * CHH, CHH)], col_v)

            def step(j, carry):
                pltpu.async_copy(h_hbm.at[row_v.at[j]], gbufa, sga).wait()
                pltpu.sync_copy(gbufa, acc.at[col_v.at[j]], add=True)
                return carry

            lax.fori_loop(0, CHH, step, 0)

        plsc.subcore_barrier()
        pltpu.sync_copy(acc.at[pl.ds(ss * TROWS, TROWS)],
                        out_hbm.at[cc, pl.ds(ss * TROWS, TROWS)])

    return agg


_agg128 = _make_agg(128)
_agg64 = _make_agg(64)


def _tc1_body(degp, x, w1, h1p_o, dinv_o):
    deg = degp[0, :, 0:1] + degp[1, :, 0:1] + 1.0
    dinv = lax.rsqrt(deg)
    h = jnp.dot(x[...], w1[...], preferred_element_type=jnp.float32)
    h1p_o[...] = h * dinv
    dinv_o[...] = dinv


def _tc2_body(p, h1p, dinv, b1, w2, h2p_o):
    s = p[0] + p[1] + h1p[...]
    o1 = jnp.maximum(dinv[...] * s + b1[...], 0.0)
    h2p_o[...] = dinv[...] * jnp.dot(o1, w2[...],
                                     preferred_element_type=jnp.float32)


def _tc3_body(p2, h2p, dinv, b2, out_o):
    out_o[...] = dinv[...] * (p2[0] + p2[1] + h2p[...]) + b2[...]


_tc1 = pl.pallas_call(
    _tc1_body,
    out_shape=[
        jax.ShapeDtypeStruct((NPAD, 128), jnp.float32),
        jax.ShapeDtypeStruct((NPAD, 1), jnp.float32),
    ],
)

_tc2 = pl.pallas_call(
    _tc2_body,
    out_shape=jax.ShapeDtypeStruct((NPAD, 64), jnp.float32),
)

_tc3 = pl.pallas_call(
    _tc3_body,
    out_shape=jax.ShapeDtypeStruct((NPAD, 64), jnp.float32),
)


def kernel(x, edge_index, W1, b1, W2, b2):
    ei = edge_index.astype(jnp.int32)
    pad = EPAD - ei.shape[1]
    row = jnp.concatenate([ei[0], jnp.zeros((pad,), jnp.int32)])
    col = jnp.concatenate([ei[1], jnp.full((pad,), N, jnp.int32)])
    er128 = row.reshape(NW, EPER // 128, 128)
    ec128 = col.reshape(NW, EPER // 128, 128)

    x_pad = jnp.zeros((NPAD, 128), jnp.float32).at[:N].set(x)
    ones_kw = jnp.ones((KDEG, DEGW), jnp.float32)
    zz16 = jnp.zeros((NPAD, DEGW), jnp.float32)
    zk128 = jnp.zeros((128, 128), jnp.float32)
    zk64 = jnp.zeros((128, 64), jnp.float32)
    b1r = b1.reshape(1, -1)
    b2r = b2.reshape(1, -1)

    degp = _deg_kernel(ec128, ones_kw, zz16)
    h1p, dinv = _tc1(degp, x_pad, W1)
    p1 = _agg128(h1p, er128, ec128, zk128)
    h2p = _tc2(p1, h1p, dinv, b1r, W2)
    p2 = _agg64(h2p, er128, ec128, zk64)
    out = _tc3(p2, h2p, dinv, b2r)
    return out[:N]


# speedup vs baseline: 2.7295x; 2.7295x over previous
import functools

import jax
import jax.numpy as jnp
from jax import lax
from jax.experimental import pallas as pl
from jax.experimental.pallas import tpu as pltpu
from jax.experimental.pallas import tpu_sc as plsc

N = 10000
NPAD = 10240
NW = 32
EPER = 10240
EPAD = NW * EPER
TROWS = NPAD // 16
DEGW = 16
KDEG = 128

_MESH = plsc.VectorSubcoreMesh(
    core_axis_name="c", subcore_axis_name="s", num_cores=2, num_subcores=16
)

_SC_PARAMS = pltpu.CompilerParams(use_tc_tiling_on_sc=False)


@functools.partial(
    pl.kernel,
    out_type=jax.ShapeDtypeStruct((2, NPAD, DEGW), jnp.float32),
    mesh=_MESH,
    scratch_types=[
        pltpu.VMEM((EPER // KDEG, KDEG), jnp.int32),
        pltpu.VMEM((KDEG, DEGW), jnp.float32),
        pltpu.VMEM_SHARED((NPAD, DEGW), jnp.float32),
    ],
    compiler_params=_SC_PARAMS,
)
def _deg_kernel(ec_hbm, ones_hbm, zz_hbm, out_hbm, col_v, obuf, acc):
    cc = lax.axis_index("c")
    ss = lax.axis_index("s")
    wid = cc * 16 + ss
    pltpu.sync_copy(zz_hbm.at[pl.ds(ss * TROWS, TROWS)],
                    acc.at[pl.ds(ss * TROWS, TROWS)])
    pltpu.sync_copy(ones_hbm, obuf)
    pltpu.sync_copy(ec_hbm.at[wid], col_v)
    plsc.subcore_barrier()

    def step(j, carry):
        pltpu.sync_copy(obuf, acc.at[col_v.at[j]], add=True)
        return carry

    lax.fori_loop(0, EPER // KDEG, step, 0)
    plsc.subcore_barrier()
    pltpu.sync_copy(acc.at[pl.ds(ss * TROWS, TROWS)],
                    out_hbm.at[cc, pl.ds(ss * TROWS, TROWS)])


def _make_agg(F):
    K = 128
    CH = EPER // K
    CHH = CH // 2
    NPAIR = CHH // 2

    @functools.partial(
        pl.kernel,
        out_type=jax.ShapeDtypeStruct((2, NPAD, F), jnp.float32),
        mesh=_MESH,
        scratch_types=[
            pltpu.VMEM((CHH, K), jnp.int32),
            pltpu.VMEM((CHH, K), jnp.int32),
            pltpu.VMEM((K, F), jnp.float32),
            pltpu.VMEM((K, F), jnp.float32),
            pltpu.VMEM_SHARED((NPAD, F), jnp.float32),
            pltpu.SemaphoreType.DMA,
            pltpu.SemaphoreType.DMA,
            pltpu.SemaphoreType.DMA,
            pltpu.SemaphoreType.DMA,
        ],
        compiler_params=_SC_PARAMS,
    )
    def agg(h_hbm, er_hbm, ec_hbm, zk_hbm, out_hbm, row_v, col_v, gbufa,
            gbufb, acc, sga, sgb, ssa, ssb):
        cc = lax.axis_index("c")
        ss = lax.axis_index("s")
        wid = cc * 16 + ss
        pltpu.sync_copy(zk_hbm, gbufa)
        for m in range(TROWS // K):
            pltpu.sync_copy(gbufa, acc.at[pl.ds(ss * TROWS + m * K, K)])
        plsc.subcore_barrier()

        for half in range(2):
            pltpu.sync_copy(er_hbm.at[wid, pl.ds(half * CHH, CHH)], row_v)
            pltpu.sync_copy(ec_hbm.at[wid, pl.ds(half * CHH, CHH)], col_v)
            pltpu.async_copy(h_hbm.at[row_v.at[0]], gbufa, sga)

            def pair(i, carry):
                ja = 2 * i
                jb = 2 * i + 1
                pltpu.make_async_copy(h_hbm.at[row_v.at[ja]], gbufa,
                                      sga).wait()
                pltpu.make_async_copy(gbufa, acc.at[col_v.at[ja]],
                                      ssa).start(add=True)

                @pl.when(i > 0)
                def _():
                    pltpu.make_async_copy(gbufb, acc.at[col_v.at[jb - 2]],
                                          ssb).wait()

                pltpu.async_copy(h_hbm.at[row_v.at[jb]], gbufb, sgb)
                pltpu.make_async_copy(h_hbm.at[row_v.at[jb]], gbufb,
                                      sgb).wait()
                pltpu.make_async_copy(gbufb, acc.at[col_v.at[jb]],
                                      ssb).start(add=True)
                pltpu.make_async_copy(gbufa, acc.at[col_v.at[ja]],
                                      ssa).wait()

                @pl.when(ja + 2 < CHH)
                def _():
                    pltpu.async_copy(h_hbm.at[row_v.at[ja + 2]], gbufa, sga)

                return carry

            lax.fori_loop(0, NPAIR, pair, 0)
            pltpu.make_async_copy(gbufb, acc.at[col_v.at[CHH - 1]],
                                  ssb).wait()

        plsc.subcore_barrier()
        pltpu.sync_copy(acc.at[pl.ds(ss * TROWS, TROWS)],
                        out_hbm.at[cc, pl.ds(ss * TROWS, TROWS)])

    return agg


_agg128 = _make_agg(128)
_agg64 = _make_agg(64)


def _tc1_body(degp, x, w1, h1p_o, dinv_o):
    deg = degp[0, :, 0:1] + degp[1, :, 0:1] + 1.0
    dinv = lax.rsqrt(deg)
    h = jnp.dot(x[...], w1[...], preferred_element_type=jnp.float32)
    h1p_o[...] = h * dinv
    dinv_o[...] = dinv


def _tc2_body(p, h1p, dinv, b1, w2, h2p_o):
    s = p[0] + p[1] + h1p[...]
    o1 = jnp.maximum(dinv[...] * s + b1[...], 0.0)
    h2p_o[...] = dinv[...] * jnp.dot(o1, w2[...],
                                     preferred_element_type=jnp.float32)


def _tc3_body(p2, h2p, dinv, b2, out_o):
    out_o[...] = dinv[...] * (p2[0] + p2[1] + h2p[...]) + b2[...]


_tc1 = pl.pallas_call(
    _tc1_body,
    out_shape=[
        jax.ShapeDtypeStruct((NPAD, 128), jnp.float32),
        jax.ShapeDtypeStruct((NPAD, 1), jnp.float32),
    ],
)

_tc2 = pl.pallas_call(
    _tc2_body,
    out_shape=jax.ShapeDtypeStruct((NPAD, 64), jnp.float32),
)

_tc3 = pl.pallas_call(
    _tc3_body,
    out_shape=jax.ShapeDtypeStruct((NPAD, 64), jnp.float32),
)


def kernel(x, edge_index, W1, b1, W2, b2):
    ei = edge_index.astype(jnp.int32)
    pad = EPAD - ei.shape[1]
    prange = jnp.arange(pad, dtype=jnp.int32)
    row = jnp.concatenate([ei[0], prange % N])
    col = jnp.concatenate([ei[1], N + prange % (NPAD - N)])
    er128 = row.reshape(NW, EPER // 128, 128)
    ec128 = col.reshape(NW, EPER // 128, 128)

    x_pad = jnp.zeros((NPAD, 128), jnp.float32).at[:N].set(x)
    ones_kw = jnp.ones((KDEG, DEGW), jnp.float32)
    zz16 = jnp.zeros((NPAD, DEGW), jnp.float32)
    zk128 = jnp.zeros((128, 128), jnp.float32)
    zk64 = jnp.zeros((128, 64), jnp.float32)
    b1r = b1.reshape(1, -1)
    b2r = b2.reshape(1, -1)

    degp = _deg_kernel(ec128, ones_kw, zz16)
    h1p, dinv = _tc1(degp, x_pad, W1)
    p1 = _agg128(h1p, er128, ec128, zk128)
    h2p = _tc2(p1, h1p, dinv, b1r, W2)
    p2 = _agg64(h2p, er128, ec128, zk64)
    out = _tc3(p2, h2p, dinv, b2r)
    return out[:N]


# speedup vs baseline: 2.7496x; 1.0074x over previous
import functools

import jax
import jax.numpy as jnp
from jax import lax
from jax.experimental import pallas as pl
from jax.experimental.pallas import tpu as pltpu
from jax.experimental.pallas import tpu_sc as plsc

N = 10000
NPAD = 10240
NW = 32
EPER = 10240
EPAD = NW * EPER
TROWS = NPAD // 16
DEGW = 16
KDEG = 128

_MESH = plsc.VectorSubcoreMesh(
    core_axis_name="c", subcore_axis_name="s", num_cores=2, num_subcores=16
)

_SC_PARAMS = pltpu.CompilerParams(use_tc_tiling_on_sc=False)


@functools.partial(
    pl.kernel,
    out_type=jax.ShapeDtypeStruct((2, NPAD, DEGW), jnp.float32),
    mesh=_MESH,
    scratch_types=[
        pltpu.VMEM((EPER // KDEG, KDEG), jnp.int32),
        pltpu.VMEM((KDEG, DEGW), jnp.float32),
        pltpu.VMEM_SHARED((NPAD, DEGW), jnp.float32),
    ],
    compiler_params=_SC_PARAMS,
)
def _deg_kernel(ec_hbm, ones_hbm, zz_hbm, out_hbm, col_v, obuf, acc):
    cc = lax.axis_index("c")
    ss = lax.axis_index("s")
    wid = cc * 16 + ss
    pltpu.sync_copy(zz_hbm.at[pl.ds(ss * TROWS, TROWS)],
                    acc.at[pl.ds(ss * TROWS, TROWS)])
    pltpu.sync_copy(ones_hbm, obuf)
    pltpu.sync_copy(ec_hbm.at[wid], col_v)
    plsc.subcore_barrier()

    def step(j, carry):
        pltpu.sync_copy(obuf, acc.at[col_v.at[j]], add=True)
        return carry

    lax.fori_loop(0, EPER // KDEG, step, 0)
    plsc.subcore_barrier()
    pltpu.sync_copy(acc.at[pl.ds(ss * TROWS, TROWS)],
                    out_hbm.at[cc, pl.ds(ss * TROWS, TROWS)])


def _make_agg(F):
    K = 128
    CH = EPER // K
    CHH = CH // 2
    NPAIR = CHH // 2

    @functools.partial(
        pl.kernel,
        out_type=jax.ShapeDtypeStruct((2, NPAD, F), jnp.float32),
        mesh=_MESH,
        scratch_types=[
            pltpu.VMEM((CHH, K), jnp.int32),
            pltpu.VMEM((CHH, K), jnp.int32),
            pltpu.VMEM((K, F), jnp.float32),
            pltpu.VMEM((K, F), jnp.float32),
            pltpu.VMEM_SHARED((NPAD, F), jnp.float32),
            pltpu.SemaphoreType.DMA,
            pltpu.SemaphoreType.DMA,
            pltpu.SemaphoreType.DMA,
            pltpu.SemaphoreType.DMA,
        ],
        compiler_params=_SC_PARAMS,
    )
    def agg(h_hbm, er_hbm, ec_hbm, zk_hbm, out_hbm, row_v, col_v, gbufa,
            gbufb, acc, sga, sgb, ssa, ssb):
        cc = lax.axis_index("c")
        ss = lax.axis_index("s")
        wid = cc * 16 + ss
        pltpu.sync_copy(zk_hbm, gbufa)
        for m in range(TROWS // K):
            pltpu.sync_copy(gbufa, acc.at[pl.ds(ss * TROWS + m * K, K)])
        plsc.subcore_barrier()

        for half in range(2):
            pltpu.sync_copy(er_hbm.at[wid, pl.ds(half * CHH, CHH)], row_v)
            pltpu.sync_copy(ec_hbm.at[wid, pl.ds(half * CHH, CHH)], col_v)
            pltpu.async_copy(h_hbm.at[row_v.at[0]], gbufa, sga)

            def pair(i, carry):
                ja = 2 * i
                jb = 2 * i + 1
                pltpu.make_async_copy(h_hbm.at[row_v.at[ja]], gbufa,
                                      sga).wait()
                pltpu.make_async_copy(gbufa, acc.at[col_v.at[ja]],
                                      ssa).start(add=True)

                @pl.when(i > 0)
                def _():
                    pltpu.make_async_copy(gbufb, acc.at[col_v.at[jb - 2]],
                                          ssb).wait()

                pltpu.async_copy(h_hbm.at[row_v.at[jb]], gbufb, sgb)
                pltpu.make_async_copy(h_hbm.at[row_v.at[jb]], gbufb,
                                      sgb).wait()
                pltpu.make_async_copy(gbufb, acc.at[col_v.at[jb]],
                                      ssb).start(add=True)
                pltpu.make_async_copy(gbufa, acc.at[col_v.at[ja]],
                                      ssa).wait()

                @pl.when(ja + 2 < CHH)
                def _():
                    pltpu.async_copy(h_hbm.at[row_v.at[ja + 2]], gbufa, sga)

                return carry

            lax.fori_loop(0, NPAIR, pair, 0)
            pltpu.make_async_copy(gbufb, acc.at[col_v.at[CHH - 1]],
                                  ssb).wait()

        plsc.subcore_barrier()
        pltpu.sync_copy(acc.at[pl.ds(ss * TROWS, TROWS)],
                        out_hbm.at[cc, pl.ds(ss * TROWS, TROWS)])

    return agg


_agg128 = _make_agg(128)
_agg64 = _make_agg(64)


def _tc1_body(degp, x, w1, h1p_o, dinv_o):
    deg = degp[0, 0:N, 0:1] + degp[1, 0:N, 0:1] + 1.0
    dinv = lax.rsqrt(deg)
    h = jnp.dot(x[...], w1[...], preferred_element_type=jnp.float32)
    h1p_o[...] = h * dinv
    dinv_o[...] = dinv


def _tc2_body(p, h1p, dinv, b1, w2, h2p_o):
    s = p[0, 0:N, :] + p[1, 0:N, :] + h1p[...]
    o1 = jnp.maximum(dinv[...] * s + b1[...], 0.0)
    h2p_o[...] = dinv[...] * jnp.dot(o1, w2[...],
                                     preferred_element_type=jnp.float32)


def _tc3_body(p2, h2p, dinv, b2, out_o):
    out_o[...] = dinv[...] * (p2[0, 0:N, :] + p2[1, 0:N, :] + h2p[...]) \
        + b2[...]


_tc1 = pl.pallas_call(
    _tc1_body,
    out_shape=[
        jax.ShapeDtypeStruct((N, 128), jnp.float32),
        jax.ShapeDtypeStruct((N, 1), jnp.float32),
    ],
)

_tc2 = pl.pallas_call(
    _tc2_body,
    out_shape=jax.ShapeDtypeStruct((N, 64), jnp.float32),
)

_tc3 = pl.pallas_call(
    _tc3_body,
    out_shape=jax.ShapeDtypeStruct((N, 64), jnp.float32),
)


def kernel(x, edge_index, W1, b1, W2, b2):
    ei = edge_index.astype(jnp.int32)
    pad = EPAD - ei.shape[1]
    prange = jnp.arange(pad, dtype=jnp.int32)
    row = jnp.concatenate([ei[0], prange % N])
    col = jnp.concatenate([ei[1], N + prange % (NPAD - N)])
    er128 = row.reshape(NW, EPER // 128, 128)
    ec128 = col.reshape(NW, EPER // 128, 128)

    ones_kw = jnp.ones((KDEG, DEGW), jnp.float32)
    zz16 = jnp.zeros((NPAD, DEGW), jnp.float32)
    zk128 = jnp.zeros((128, 128), jnp.float32)
    zk64 = jnp.zeros((128, 64), jnp.float32)
    b1r = b1.reshape(1, -1)
    b2r = b2.reshape(1, -1)

    degp = _deg_kernel(ec128, ones_kw, zz16)
    h1p, dinv = _tc1(degp, x, W1)
    p1 = _agg128(h1p, er128, ec128, zk128)
    h2p = _tc2(p1, h1p, dinv, b1r, W2)
    p2 = _agg64(h2p, er128, ec128, zk64)
    return _tc3(p2, h2p, dinv, b2r)


# speedup vs baseline: 2.8529x; 1.0376x over previous
import functools

import jax
import jax.numpy as jnp
from jax import lax
from jax.experimental import pallas as pl
from jax.experimental.pallas import tpu as pltpu
from jax.experimental.pallas import tpu_sc as plsc

N = 10000
NPAD = 10240
NW = 32
EPER = 10240
EPAD = NW * EPER
TROWS = NPAD // 16
DEGW = 16
KDEG = 128

_MESH = plsc.VectorSubcoreMesh(
    core_axis_name="c", subcore_axis_name="s", num_cores=2, num_subcores=16
)

_SC_PARAMS = pltpu.CompilerParams(use_tc_tiling_on_sc=False)


@functools.partial(
    pl.kernel,
    out_type=jax.ShapeDtypeStruct((2, NPAD, DEGW), jnp.float32),
    mesh=_MESH,
    scratch_types=[
        pltpu.VMEM((EPER // KDEG, KDEG), jnp.int32),
        pltpu.VMEM((KDEG, DEGW), jnp.float32),
        pltpu.VMEM_SHARED((NPAD, DEGW), jnp.float32),
    ],
    compiler_params=_SC_PARAMS,
)
def _deg_kernel(ec_hbm, ones_hbm, zz_hbm, out_hbm, col_v, obuf, acc):
    cc = lax.axis_index("c")
    ss = lax.axis_index("s")
    wid = cc * 16 + ss
    pltpu.sync_copy(zz_hbm.at[pl.ds(ss * TROWS, TROWS)],
                    acc.at[pl.ds(ss * TROWS, TROWS)])
    pltpu.sync_copy(ones_hbm, obuf)
    pltpu.sync_copy(ec_hbm.at[wid], col_v)
    plsc.subcore_barrier()

    def step(j, carry):
        pltpu.sync_copy(obuf, acc.at[col_v.at[j]], add=True)
        return carry

    lax.fori_loop(0, EPER // KDEG, step, 0)
    plsc.subcore_barrier()
    pltpu.sync_copy(acc.at[pl.ds(ss * TROWS, TROWS)],
                    out_hbm.at[cc, pl.ds(ss * TROWS, TROWS)])


def _make_agg(F):
    K = 128
    CH = EPER // K
    CHH = CH // 2
    NPAIR = CHH // 2

    @functools.partial(
        pl.kernel,
        out_type=jax.ShapeDtypeStruct((2, NPAD, F), jnp.float32),
        mesh=_MESH,
        scratch_types=[
            pltpu.VMEM((CHH, K), jnp.int32),
            pltpu.VMEM((CHH, K), jnp.int32),
            pltpu.VMEM((K, F), jnp.float32),
            pltpu.VMEM((K, F), jnp.float32),
            pltpu.VMEM_SHARED((NPAD, F), jnp.float32),
            pltpu.SemaphoreType.DMA,
            pltpu.SemaphoreType.DMA,
            pltpu.SemaphoreType.DMA,
            pltpu.SemaphoreType.DMA,
        ],
        compiler_params=_SC_PARAMS,
    )
    def agg(h_hbm, er_hbm, ec_hbm, zk_hbm, out_hbm, row_v, col_v, gbufa,
            gbufb, acc, sga, sgb, ssa, ssb):
        cc = lax.axis_index("c")
        ss = lax.axis_index("s")
        wid = cc * 16 + ss
        pltpu.sync_copy(zk_hbm, gbufa)
        for m in range(TROWS // K):
            pltpu.sync_copy(gbufa, acc.at[pl.ds(ss * TROWS + m * K, K)])
        plsc.subcore_barrier()

        for half in range(2):
            pltpu.sync_copy(er_hbm.at[wid, pl.ds(half * CHH, CHH)], row_v)
            pltpu.sync_copy(ec_hbm.at[wid, pl.ds(half * CHH, CHH)], col_v)
            pltpu.async_copy(h_hbm.at[row_v.at[0]], gbufa, sga)

            def pair(i, carry):
                ja = 2 * i
                jb = 2 * i + 1
                pltpu.make_async_copy(h_hbm.at[row_v.at[ja]], gbufa,
                                      sga).wait()
                pltpu.make_async_copy(gbufa, acc.at[col_v.at[ja]],
                                      ssa).start(add=True)

                @pl.when(i > 0)
                def _():
                    pltpu.make_async_copy(gbufb, acc.at[col_v.at[jb - 2]],
                                          ssb).wait()

                pltpu.async_copy(h_hbm.at[row_v.at[jb]], gbufb, sgb)
                pltpu.make_async_copy(h_hbm.at[row_v.at[jb]], gbufb,
                                      sgb).wait()
                pltpu.make_async_copy(gbufb, acc.at[col_v.at[jb]],
                                      ssb).start(add=True)
                pltpu.make_async_copy(gbufa, acc.at[col_v.at[ja]],
                                      ssa).wait()

                @pl.when(ja + 2 < CHH)
                def _():
                    pltpu.async_copy(h_hbm.at[row_v.at[ja + 2]], gbufa, sga)

                return carry

            lax.fori_loop(0, NPAIR, pair, 0)
            pltpu.make_async_copy(gbufb, acc.at[col_v.at[CHH - 1]],
                                  ssb).wait()

        plsc.subcore_barrier()
        pltpu.sync_copy(acc.at[pl.ds(ss * TROWS, TROWS)],
                        out_hbm.at[cc, pl.ds(ss * TROWS, TROWS)])

    return agg


_agg128 = _make_agg(128)


def _make_agg_staged(F):
    K = 128
    CH = EPER // K
    CHH = CH // 2
    NPAIR = CHH // 2
    HROWS = N
    SROWS = HROWS // 16

    @functools.partial(
        pl.kernel,
        out_type=jax.ShapeDtypeStruct((2, NPAD, F), jnp.float32),
        mesh=_MESH,
        scratch_types=[
            pltpu.VMEM((CHH, K), jnp.int32),
            pltpu.VMEM((CHH, K), jnp.int32),
            pltpu.VMEM((K, F), jnp.float32),
            pltpu.VMEM((K, F), jnp.float32),
            pltpu.VMEM_SHARED((NPAD, F), jnp.float32),
            pltpu.VMEM_SHARED((HROWS, F), jnp.float32),
            pltpu.SemaphoreType.DMA,
            pltpu.SemaphoreType.DMA,
            pltpu.SemaphoreType.DMA,
            pltpu.SemaphoreType.DMA,
        ],
        compiler_params=_SC_PARAMS,
    )
    def agg(h_hbm, er_hbm, ec_hbm, zk_hbm, out_hbm, row_v, col_v, gbufa,
            gbufb, acc, hstage, sga, sgb, ssa, ssb):
        cc = lax.axis_index("c")
        ss = lax.axis_index("s")
        wid = cc * 16 + ss
        pltpu.sync_copy(zk_hbm, gbufa)
        for m in range(TROWS // K):
            pltpu.sync_copy(gbufa, acc.at[pl.ds(ss * TROWS + m * K, K)])
        pltpu.sync_copy(h_hbm.at[pl.ds(ss * SROWS, SROWS)],
                        hstage.at[pl.ds(ss * SROWS, SROWS)])
        plsc.subcore_barrier()

        for half in range(2):
            pltpu.sync_copy(er_hbm.at[wid, pl.ds(half * CHH, CHH)], row_v)
            pltpu.sync_copy(ec_hbm.at[wid, pl.ds(half * CHH, CHH)], col_v)
            pltpu.async_copy(hstage.at[row_v.at[0]], gbufa, sga)

            def pair(i, carry):
                ja = 2 * i
                jb = 2 * i + 1
                pltpu.make_async_copy(hstage.at[row_v.at[ja]], gbufa,
                                      sga).wait()
                pltpu.make_async_copy(gbufa, acc.at[col_v.at[ja]],
                                      ssa).start(add=True)

                @pl.when(i > 0)
                def _():
                    pltpu.make_async_copy(gbufb, acc.at[col_v.at[jb - 2]],
                                          ssb).wait()

                pltpu.async_copy(hstage.at[row_v.at[jb]], gbufb, sgb)
                pltpu.make_async_copy(hstage.at[row_v.at[jb]], gbufb,
                                      sgb).wait()
                pltpu.make_async_copy(gbufb, acc.at[col_v.at[jb]],
                                      ssb).start(add=True)
                pltpu.make_async_copy(gbufa, acc.at[col_v.at[ja]],
                                      ssa).wait()

                @pl.when(ja + 2 < CHH)
                def _():
                    pltpu.async_copy(hstage.at[row_v.at[ja + 2]], gbufa, sga)

                return carry

            lax.fori_loop(0, NPAIR, pair, 0)
            pltpu.make_async_copy(gbufb, acc.at[col_v.at[CHH - 1]],
                                  ssb).wait()

        plsc.subcore_barrier()
        pltpu.sync_copy(acc.at[pl.ds(ss * TROWS, TROWS)],
                        out_hbm.at[cc, pl.ds(ss * TROWS, TROWS)])

    return agg


_agg64 = _make_agg_staged(64)


def _tc1_body(degp, x, w1, h1p_o, dinv_o):
    deg = degp[0, 0:N, 0:1] + degp[1, 0:N, 0:1] + 1.0
    dinv = lax.rsqrt(deg)
    h = jnp.dot(x[...], w1[...], preferred_element_type=jnp.float32)
    h1p_o[...] = h * dinv
    dinv_o[...] = dinv


def _tc2_body(p, h1p, dinv, b1, w2, h2p_o):
    s = p[0, 0:N, :] + p[1, 0:N, :] + h1p[...]
    o1 = jnp.maximum(dinv[...] * s + b1[...], 0.0)
    h2p_o[...] = dinv[...] * jnp.dot(o1, w2[...],
                                     preferred_element_type=jnp.float32)


def _tc3_body(p2, h2p, dinv, b2, out_o):
    out_o[...] = dinv[...] * (p2[0, 0:N, :] + p2[1, 0:N, :] + h2p[...]) \
        + b2[...]


_tc1 = pl.pallas_call(
    _tc1_body,
    out_shape=[
        jax.ShapeDtypeStruct((N, 128), jnp.float32),
        jax.ShapeDtypeStruct((N, 1), jnp.float32),
    ],
)

_tc2 = pl.pallas_call(
    _tc2_body,
    out_shape=jax.ShapeDtypeStruct((N, 64), jnp.float32),
)

_tc3 = pl.pallas_call(
    _tc3_body,
    out_shape=jax.ShapeDtypeStruct((N, 64), jnp.float32),
)


def kernel(x, edge_index, W1, b1, W2, b2):
    ei = edge_index.astype(jnp.int32)
    pad = EPAD - ei.shape[1]
    prange = jnp.arange(pad, dtype=jnp.int32)
    row = jnp.concatenate([ei[0], prange % N])
    col = jnp.concatenate([ei[1], N + prange % (NPAD - N)])
    er128 = row.reshape(NW, EPER // 128, 128)
    ec128 = col.reshape(NW, EPER // 128, 128)

    ones_kw = jnp.ones((KDEG, DEGW), jnp.float32)
    zz16 = jnp.zeros((NPAD, DEGW), jnp.float32)
    zk128 = jnp.zeros((128, 128), jnp.float32)
    zk64 = jnp.zeros((128, 64), jnp.float32)
    b1r = b1.reshape(1, -1)
    b2r = b2.reshape(1, -1)

    degp = _deg_kernel(ec128, ones_kw, zz16)
    h1p, dinv = _tc1(degp, x, W1)
    p1 = _agg128(h1p, er128, ec128, zk128)
    h2p = _tc2(p1, h1p, dinv, b1r, W2)
    p2 = _agg64(h2p, er128, ec128, zk64)
    return _tc3(p2, h2p, dinv, b2r)
